# baseline (speedup 1.0000x reference)
import functools

import jax
import jax.numpy as jnp
from jax import lax
from jax.experimental import pallas as pl
from jax.experimental.pallas import tpu as pltpu
from jax.experimental.pallas import tpu_sc as plsc

LATENT = 100
NUM_EMB = 100
BN = 2048
CHUNK = 128
ROW = 128


def _idx_block(x_ref, e_ref, xsq_ref, esq_ref, idx_ref):
    xb = x_ref[...]
    eb = e_ref[...]
    dt = lax.dot_general(eb, xb, (((1,), (1,)), ((), ())),
                         preferred_element_type=jnp.float32)
    x_sq = jnp.reshape(xsq_ref[...], (1, -1))
    e_sq = esq_ref[...]
    d2 = (x_sq - 2.0 * dt) + e_sq
    dists = jnp.sqrt(jnp.maximum(d2, 0.0))
    dmin = jnp.min(dists, axis=0, keepdims=True)
    iota_k = lax.broadcasted_iota(jnp.int32, dists.shape, 0)
    idx = jnp.min(jnp.where(dists == dmin, iota_k, NUM_EMB), axis=0)
    idx_ref[...] = jnp.reshape(idx, idx_ref.shape)


def _tc_indices(flat, embeddings, x_sq, e_sq, n):
    nb = n // BN
    return pl.pallas_call(
        _idx_block,
        grid=(nb,),
        in_specs=[
            pl.BlockSpec((BN, LATENT), lambda i: (i, 0)),
            pl.BlockSpec((NUM_EMB, LATENT), lambda i: (0, 0)),
            pl.BlockSpec((1, 1, BN), lambda i: (i, 0, 0)),
            pl.BlockSpec((NUM_EMB, 1), lambda i: (0, 0)),
        ],
        out_specs=pl.BlockSpec((1, 1, BN), lambda i: (i, 0, 0)),
        out_shape=jax.ShapeDtypeStruct((nb, 1, BN), jnp.int32),
    )(flat, embeddings, x_sq, e_sq)


def _make_sc_gather(n):
    info = plsc.get_sparse_core_info()
    nw = info.num_cores * info.num_subcores
    b_per_w = n // nw
    n_chunks = b_per_w // CHUNK
    n_groups = CHUNK // 16

    @functools.partial(
        pl.kernel,
        out_type=jax.ShapeDtypeStruct((n * ROW,), jnp.float32),
        mesh=plsc.VectorSubcoreMesh(core_axis_name="c", subcore_axis_name="s"),
        compiler_params=pltpu.CompilerParams(needs_layout_passes=False),
        scratch_types=[
            pltpu.VMEM((b_per_w,), jnp.int32),
            pltpu.VMEM((NUM_EMB * ROW,), jnp.float32),
            pltpu.VMEM((CHUNK * ROW,), jnp.float32),
            pltpu.VMEM((CHUNK * ROW,), jnp.float32),
            pltpu.SemaphoreType.DMA,
            pltpu.SemaphoreType.DMA,
        ],
    )
    def sc_gather(table_h, idx_h, out_h, idx_v, table_v, buf0, buf1,
                  sem0, sem1):
        wid = lax.axis_index("s") * info.num_cores + lax.axis_index("c")
        base = wid * b_per_w
        pltpu.sync_copy(table_h, table_v)
        pltpu.sync_copy(idx_h.at[pl.ds(base, b_per_w)], idx_v)
        bufs = (buf0, buf1)
        sems = (sem0, sem1)
        stores = [None, None]
        lane = lax.iota(jnp.int32, 16)

        for j in range(n_chunks):
            buf = bufs[j % 2]
            if stores[j % 2] is not None:
                stores[j % 2].wait()
            def gbody(g, _, buf=buf, j=j):
                rows = idx_v[pl.ds(j * CHUNK + g * 16, 16)]
                src = rows * ROW
                dst = (lane + g * 16) * ROW
                for wb in range(0, ROW, 16):
                    vals = []
                    for u in range(16):
                        wvec = (wb + u + lane) & (ROW - 1)
                        vals.append(
                            (wvec, plsc.load_gather(table_v, [src + wvec])))
                    for wvec, v in vals:
                        plsc.store_scatter(buf, [dst + wvec], v)
                return _

            lax.fori_loop(0, n_groups, gbody, 0)
            stores[j % 2] = pltpu.async_copy(
                buf, out_h.at[pl.ds((base + j * CHUNK) * ROW, CHUNK * ROW)],
                sems[j % 2])
        stores[0].wait()
        stores[1].wait()

    return sc_gather


@jax.jit
def _vq(inputs, embeddings):
    shape = inputs.shape
    flat = jnp.reshape(inputs, (-1, LATENT))
    n = flat.shape[0]
    x_sq = jnp.reshape(jnp.sum(flat * flat, axis=1), (n // BN, 1, BN))
    e_sq = jnp.reshape(jnp.sum(embeddings * embeddings, axis=1), (NUM_EMB, 1))
    idx = _tc_indices(flat, embeddings, x_sq, e_sq, n)
    table = jnp.reshape(jnp.pad(embeddings, ((0, 0), (0, ROW - LATENT))),
                        (NUM_EMB * ROW,))
    out = _make_sc_gather(n)(table, jnp.reshape(idx, (n,)))
    return jnp.reshape(jnp.reshape(out, (n, ROW))[:, :LATENT], shape)


def kernel(inputs, embeddings):
    return _vq(inputs, embeddings)


# speedup vs baseline: 1.0018x; 1.0018x over previous
import functools

import jax
import jax.numpy as jnp
from jax import lax
from jax.experimental import pallas as pl
from jax.experimental.pallas import tpu as pltpu
from jax.experimental.pallas import tpu_sc as plsc

LATENT = 100
NUM_EMB = 100
BN = 2048
CHUNK = 128
ROW = 128


def _idx_block(x_ref, e_ref, xsq_ref, esq_ref, idx_ref):
    xb = x_ref[...]
    eb = e_ref[...]
    dt = lax.dot_general(eb, xb, (((1,), (1,)), ((), ())),
                         preferred_element_type=jnp.float32)
    x_sq = jnp.reshape(xsq_ref[...], (1, -1))
    e_sq = esq_ref[...]
    d2 = (x_sq - 2.0 * dt) + e_sq
    dists = jnp.sqrt(jnp.maximum(d2, 0.0))
    dmin = jnp.min(dists, axis=0, keepdims=True)
    iota_k = lax.broadcasted_iota(jnp.int32, dists.shape, 0)
    idx = jnp.min(jnp.where(dists == dmin, iota_k, NUM_EMB), axis=0)
    idx_ref[...] = jnp.reshape(idx, idx_ref.shape)


def _tc_indices(flat, embeddings, x_sq, e_sq, n):
    nb = n // BN
    return pl.pallas_call(
        _idx_block,
        grid=(nb,),
        in_specs=[
            pl.BlockSpec((BN, LATENT), lambda i: (i, 0)),
            pl.BlockSpec((NUM_EMB, LATENT), lambda i: (0, 0)),
            pl.BlockSpec((1, 1, BN), lambda i: (i, 0, 0)),
            pl.BlockSpec((NUM_EMB, 1), lambda i: (0, 0)),
        ],
        out_specs=pl.BlockSpec((1, 1, BN), lambda i: (i, 0, 0)),
        out_shape=jax.ShapeDtypeStruct((nb, 1, BN), jnp.int32),
    )(flat, embeddings, x_sq, e_sq)


def _make_sc_gather(n):
    info = plsc.get_sparse_core_info()
    nw = info.num_cores * info.num_subcores
    b_per_w = n // nw
    n_chunks = b_per_w // CHUNK
    n_groups = CHUNK // 16

    @functools.partial(
        pl.kernel,
        out_type=jax.ShapeDtypeStruct((n * ROW,), jnp.float32),
        mesh=plsc.VectorSubcoreMesh(core_axis_name="c", subcore_axis_name="s"),
        compiler_params=pltpu.CompilerParams(needs_layout_passes=False),
        scratch_types=[
            pltpu.VMEM((b_per_w,), jnp.int32),
            pltpu.VMEM((NUM_EMB * ROW,), jnp.float32),
            pltpu.VMEM((CHUNK * ROW,), jnp.float32),
            pltpu.VMEM((CHUNK * ROW,), jnp.float32),
            pltpu.SemaphoreType.DMA,
            pltpu.SemaphoreType.DMA,
        ],
    )
    def sc_gather(table_h, idx_h, out_h, idx_v, table_v, buf0, buf1,
                  sem0, sem1):
        wid = lax.axis_index("s") * info.num_cores + lax.axis_index("c")
        base = wid * b_per_w
        pltpu.sync_copy(table_h, table_v)
        pltpu.sync_copy(idx_h.at[pl.ds(base, b_per_w)], idx_v)
        bufs = (buf0, buf1)
        sems = (sem0, sem1)
        stores = [None, None]
        lane = lax.iota(jnp.int32, 16)

        for j in range(n_chunks):
            buf = bufs[j % 2]
            if stores[j % 2] is not None:
                stores[j % 2].wait()
            def gbody(g, _, buf=buf, j=j):
                rows = idx_v[pl.ds(j * CHUNK + g * 16, 16)]
                src = rows * ROW
                dst = (lane + g * 16) * ROW
                for wb in range(0, ROW, 8):
                    vals = []
                    for u in range(8):
                        wvec = (wb + u + lane) & (ROW - 1)
                        vals.append(
                            (wvec, plsc.load_gather(table_v, [src + wvec])))
                    for wvec, v in vals:
                        plsc.store_scatter(buf, [dst + wvec], v)
                return _

            lax.fori_loop(0, n_groups, gbody, 0)
            stores[j % 2] = pltpu.async_copy(
                buf, out_h.at[pl.ds((base + j * CHUNK) * ROW, CHUNK * ROW)],
                sems[j % 2])
        stores[0].wait()
        stores[1].wait()

    return sc_gather


@jax.jit
def _vq(inputs, embeddings):
    shape = inputs.shape
    flat = jnp.reshape(inputs, (-1, LATENT))
    n = flat.shape[0]
    x_sq = jnp.reshape(jnp.sum(flat * flat, axis=1), (n // BN, 1, BN))
    e_sq = jnp.reshape(jnp.sum(embeddings * embeddings, axis=1), (NUM_EMB, 1))
    idx = _tc_indices(flat, embeddings, x_sq, e_sq, n)
    table = jnp.reshape(jnp.pad(embeddings, ((0, 0), (0, ROW - LATENT))),
                        (NUM_EMB * ROW,))
    out = _make_sc_gather(n)(table, jnp.reshape(idx, (n,)))
    return jnp.reshape(jnp.reshape(out, (n, ROW))[:, :LATENT], shape)


def kernel(inputs, embeddings):
    return _vq(inputs, embeddings)
